# baseline (speedup 1.0000x reference)
import functools

import jax
import jax.numpy as jnp
from jax import lax
from jax.experimental import pallas as pl
from jax.experimental.pallas import tpu as pltpu
from jax.experimental.pallas import tpu_sc as plsc

NUM_CORES = 2
NUM_SUBCORES = 16
NW = NUM_CORES * NUM_SUBCORES
CH = 128
FAST_SHARE = 0.4975


def _sc_aggregate(x, idx, *, n_pad, d, total_chunks):
    rpt = n_pad // NUM_SUBCORES
    k0 = int(total_chunks * FAST_SHARE)
    core_base = (0, k0)
    core_cnt = (k0, total_chunks - k0)
    mesh = plsc.VectorSubcoreMesh(core_axis_name="c", subcore_axis_name="s")

    @functools.partial(
        pl.kernel,
        mesh=mesh,
        out_type=(
            jax.ShapeDtypeStruct((NUM_CORES, n_pad, d), jnp.bfloat16),
            jax.ShapeDtypeStruct((NUM_CORES, n_pad), jnp.float32),
        ),
        scratch_types=[
            pltpu.VMEM_SHARED((n_pad, d), jnp.bfloat16),
            pltpu.VMEM_SHARED((NUM_SUBCORES, n_pad), jnp.float32),
            pltpu.VMEM((2, 2, CH), jnp.int32),
            pltpu.VMEM((2, CH, d), jnp.bfloat16),
            pltpu.VMEM((n_pad,), jnp.float32),
            pltpu.VMEM((NUM_SUBCORES, rpt), jnp.float32),
            pltpu.SemaphoreType.DMA((2,)),
            pltpu.SemaphoreType.DMA((2,)),
        ],
        compiler_params=pltpu.CompilerParams(needs_layout_passes=False,
                                             use_tc_tiling_on_sc=False),
    )
    def agg(x_hbm, idx_hbm, sums_hbm, cnts_hbm,
            accum_s, cnt_all, idxb, rows, cnt_v, cnt_tmp, gsem, isem):
        c = lax.axis_index("c")
        s = lax.axis_index("s")
        wid = c * NUM_SUBCORES + s

        base = jnp.where(c == 0, core_base[0], core_base[1]) + s
        kc = jnp.where(c == 0, core_cnt[0], core_cnt[1])
        n_i = (kc - s + NUM_SUBCORES - 1) // NUM_SUBCORES

        zero16 = jnp.zeros((16,), jnp.float32)

        def zstep(i, carry):
            cnt_v[pl.ds(i * 16, 16)] = zero16
            return carry

        lax.fori_loop(0, n_pad // 16, zstep, 0)

        zero32 = jnp.zeros((32,), jnp.bfloat16)

        def zstep2(i, carry):
            rows[0, i // (d // 32), pl.ds((i % (d // 32)) * 32, 32)] = zero32
            return carry

        lax.fori_loop(0, CH * d // 32, zstep2, 0)
        for blk in range(rpt // CH):
            pltpu.sync_copy(rows.at[0],
                            accum_s.at[pl.ds(s * rpt + blk * CH, CH)])
        plsc.subcore_barrier()

        one16 = jnp.ones((16,), jnp.float32)

        def off_of(i):
            return (base + i * NUM_SUBCORES) * CH

        def idxcp(i, b):
            return pltpu.make_async_copy(
                idx_hbm.at[:, pl.ds(off_of(i), CH)], idxb.at[b], isem.at[b])

        def gather(b):
            return pltpu.make_async_copy(x_hbm.at[idxb.at[b, 0]], rows.at[b],
                                         gsem.at[b])

        @pl.when(n_i > 0)
        def _():
            pltpu.sync_copy(idx_hbm.at[:, pl.ds(off_of(0), CH)], idxb.at[0])
            gather(0).start()

            @pl.when(n_i > 1)
            def _():
                idxcp(1, 1).start()

            def step(i, carry):
                b = lax.rem(i, 2)
                nb = lax.rem(i + 1, 2)

                @pl.when(i + 1 < n_i)
                def _():
                    idxcp(i + 1, nb).wait()
                    gather(nb).start()

                for j in range(CH // 16):
                    iv = idxb[b, 1, pl.ds(j * 16, 16)]
                    plsc.addupdate_scatter(cnt_v, [iv], one16)

                gather(b).wait()
                pltpu.sync_copy(rows.at[b], accum_s.at[idxb.at[b, 1]],
                                add=True)

                @pl.when(i + 2 < n_i)
                def _():
                    idxcp(i + 2, b).start()

                return carry

            lax.fori_loop(0, n_i, step, 0)

        pltpu.sync_copy(cnt_v, cnt_all.at[s])
        plsc.subcore_barrier()
        for t in range(NUM_SUBCORES):
            pltpu.sync_copy(cnt_all.at[t, pl.ds(s * rpt, rpt)], cnt_tmp.at[t])

        def cstep(k, carry):
            acc = cnt_tmp[0, pl.ds(k * 16, 16)]
            for t in range(1, NUM_SUBCORES):
                acc = acc + cnt_tmp[t, pl.ds(k * 16, 16)]
            cnt_v[pl.ds(k * 16, 16)] = acc
            return carry

        lax.fori_loop(0, rpt // 16, cstep, 0)
        pltpu.sync_copy(cnt_v.at[pl.ds(0, rpt)],
                        cnts_hbm.at[c, pl.ds(s * rpt, rpt)])

        pltpu.sync_copy(accum_s.at[pl.ds(s * rpt, rpt)],
                        sums_hbm.at[c, pl.ds(s * rpt, rpt)])

    return agg(x, idx)


_DNUMS = (((1,), (1,)), ((), ()))


def _tc_root(x, wr, bl, *, n, d, rb):
    def body(x_ref, wr_ref, bl_ref, out_ref):
        out_ref[...] = lax.dot_general(
            x_ref[...], wr_ref[...], _DNUMS,
            preferred_element_type=jnp.float32) + bl_ref[...]

    return pl.pallas_call(
        body,
        grid=(-(-n // rb),),
        in_specs=[
            pl.BlockSpec((rb, d), lambda i: (i, 0)),
            pl.BlockSpec((d, d), lambda i: (0, 0)),
            pl.BlockSpec((1, d), lambda i: (0, 0)),
        ],
        out_specs=pl.BlockSpec((rb, d), lambda i: (i, 0)),
        out_shape=jax.ShapeDtypeStruct((n, d), jnp.float32),
    )(x, wr, bl)


def _tc_combine(sums, cnts, root, wl, *, n, d, rb):
    def body(sums_ref, cnts_ref, root_ref, wl_ref, out_ref):
        ssum = (sums_ref[0].astype(jnp.float32)
                + sums_ref[1].astype(jnp.float32))
        cnt = jnp.reshape(cnts_ref[0] + cnts_ref[1], (rb, 1))
        mean = ssum / jnp.maximum(cnt, 1.0)
        acc = lax.dot_general(mean, wl_ref[...], _DNUMS,
                              preferred_element_type=jnp.float32)
        out_ref[...] = acc + root_ref[...]

    grid = (-(-n // rb),)
    return pl.pallas_call(
        body,
        grid=grid,
        in_specs=[
            pl.BlockSpec((2, rb, d), lambda i: (0, i, 0)),
            pl.BlockSpec((2, rb), lambda i: (0, i)),
            pl.BlockSpec((rb, d), lambda i: (i, 0)),
            pl.BlockSpec((d, d), lambda i: (0, 0)),
        ],
        out_specs=pl.BlockSpec((rb, d), lambda i: (i, 0)),
        out_shape=jax.ShapeDtypeStruct((n, d), jnp.float32),
    )(sums, cnts, root, wl)


def kernel(hidden_states, graph, aggl, W_l, b_l, W_r):
    x = hidden_states
    n, d = x.shape
    e = graph.shape[1]
    n_pad = -(-(n + 1) // 256) * 256

    if e % CH == 0:
        idx = graph
    else:
        pad = CH - e % CH
        idx = jnp.concatenate(
            [graph,
             jnp.stack([jnp.zeros((pad,), jnp.int32),
                        jnp.full((pad,), n, jnp.int32)])], axis=1)
    total_chunks = idx.shape[1] // CH

    root = _tc_root(x, W_r, b_l.reshape(1, d), n=n, d=d, rb=1024)
    sums, cnts = _sc_aggregate(x.astype(jnp.bfloat16), idx,
                               n_pad=n_pad, d=d, total_chunks=total_chunks)

    out = _tc_combine(sums, cnts, root, W_l, n=n, d=d, rb=1024)
    return out


# speedup vs baseline: 1.0314x; 1.0314x over previous
import functools

import jax
import jax.numpy as jnp
from jax import lax
from jax.experimental import pallas as pl
from jax.experimental.pallas import tpu as pltpu
from jax.experimental.pallas import tpu_sc as plsc

NUM_CORES = 2
NUM_SUBCORES = 16
NW = NUM_CORES * NUM_SUBCORES
CH = 128
FAST_SHARE = 0.4975


def _sc_aggregate(x, idx, *, n_pad, d, total_chunks):
    rpt = n_pad // NUM_SUBCORES
    k0 = int(total_chunks * FAST_SHARE)
    core_base = (0, k0)
    core_cnt = (k0, total_chunks - k0)
    mesh = plsc.VectorSubcoreMesh(core_axis_name="c", subcore_axis_name="s")

    @functools.partial(
        pl.kernel,
        mesh=mesh,
        out_type=(
            jax.ShapeDtypeStruct((n_pad, d), jnp.bfloat16),
            jax.ShapeDtypeStruct((n_pad, d), jnp.bfloat16),
            jax.ShapeDtypeStruct((NW, n_pad), jnp.float32),
        ),
        scratch_types=[
            pltpu.VMEM_SHARED((n_pad, d), jnp.bfloat16),
            pltpu.VMEM((2, 2, CH), jnp.int32),
            pltpu.VMEM((2, CH, d), jnp.bfloat16),
            pltpu.VMEM((n_pad,), jnp.float32),
            pltpu.SemaphoreType.DMA((2,)),
            pltpu.SemaphoreType.DMA((2,)),
        ],
        compiler_params=pltpu.CompilerParams(needs_layout_passes=False,
                                             use_tc_tiling_on_sc=False),
    )
    def agg(x_hbm, idx_hbm, sums0_hbm, sums1_hbm, cnts_hbm,
            accum_s, idxb, rows, cnt_v, gsem, isem):
        c = lax.axis_index("c")
        s = lax.axis_index("s")
        wid = c * NUM_SUBCORES + s

        base = jnp.where(c == 0, core_base[0], core_base[1]) + s
        kc = jnp.where(c == 0, core_cnt[0], core_cnt[1])
        n_i = (kc - s + NUM_SUBCORES - 1) // NUM_SUBCORES

        zero16 = jnp.zeros((16,), jnp.float32)

        def zstep(i, carry):
            cnt_v[pl.ds(i * 16, 16)] = zero16
            return carry

        lax.fori_loop(0, n_pad // 16, zstep, 0)

        zero32 = jnp.zeros((32,), jnp.bfloat16)

        def zstep2(i, carry):
            rows[0, i // (d // 32), pl.ds((i % (d // 32)) * 32, 32)] = zero32
            return carry

        lax.fori_loop(0, CH * d // 32, zstep2, 0)
        for blk in range(rpt // CH):
            pltpu.sync_copy(rows.at[0],
                            accum_s.at[pl.ds(s * rpt + blk * CH, CH)])
        plsc.subcore_barrier()

        one16 = jnp.ones((16,), jnp.float32)

        def off_of(i):
            return (base + i * NUM_SUBCORES) * CH

        def idxcp(i, b):
            return pltpu.make_async_copy(
                idx_hbm.at[:, pl.ds(off_of(i), CH)], idxb.at[b], isem.at[b])

        def gather(b):
            return pltpu.make_async_copy(x_hbm.at[idxb.at[b, 0]], rows.at[b],
                                         gsem.at[b])

        @pl.when(n_i > 0)
        def _():
            pltpu.sync_copy(idx_hbm.at[:, pl.ds(off_of(0), CH)], idxb.at[0])
            gather(0).start()

            @pl.when(n_i > 1)
            def _():
                idxcp(1, 1).start()

            def step(i, carry):
                b = lax.rem(i, 2)
                nb = lax.rem(i + 1, 2)

                @pl.when(i + 1 < n_i)
                def _():
                    idxcp(i + 1, nb).wait()
                    gather(nb).start()

                for j in range(CH // 16):
                    iv = idxb[b, 1, pl.ds(j * 16, 16)]
                    plsc.addupdate_scatter(cnt_v, [iv], one16)

                gather(b).wait()
                pltpu.sync_copy(rows.at[b], accum_s.at[idxb.at[b, 1]],
                                add=True)

                @pl.when(i + 2 < n_i)
                def _():
                    idxcp(i + 2, b).start()

                return carry

            lax.fori_loop(0, n_i, step, 0)

        pltpu.sync_copy(cnt_v, cnts_hbm.at[wid])
        plsc.subcore_barrier()

        @pl.when(c == 0)
        def _():
            pltpu.sync_copy(accum_s.at[pl.ds(s * rpt, rpt)],
                            sums0_hbm.at[pl.ds(s * rpt, rpt)])

        @pl.when(c == 1)
        def _():
            pltpu.sync_copy(accum_s.at[pl.ds(s * rpt, rpt)],
                            sums1_hbm.at[pl.ds(s * rpt, rpt)])

    return agg(x, idx)


_DNUMS = (((1,), (1,)), ((), ()))


def _tc_root(x, wr, bl, *, n, d, rb):
    def body(x_ref, wr_ref, bl_ref, out_ref):
        out_ref[...] = lax.dot_general(
            x_ref[...], wr_ref[...], _DNUMS,
            preferred_element_type=jnp.float32) + bl_ref[...]

    return pl.pallas_call(
        body,
        grid=(-(-n // rb),),
        in_specs=[
            pl.BlockSpec((rb, d), lambda i: (i, 0)),
            pl.BlockSpec((d, d), lambda i: (0, 0)),
            pl.BlockSpec((1, d), lambda i: (0, 0)),
        ],
        out_specs=pl.BlockSpec((rb, d), lambda i: (i, 0)),
        out_shape=jax.ShapeDtypeStruct((n, d), jnp.float32),
    )(x, wr, bl)


def _tc_combine(s0, s1, cnts, root, wl, *, n, d, rb):
    def body(s0_ref, s1_ref, cnts_ref, root_ref, wl_ref, out_ref):
        v0 = jnp.reshape(s0_ref[...], (rb, d)).astype(jnp.float32)
        v1 = jnp.reshape(s1_ref[...], (rb, d)).astype(jnp.float32)
        ssum = v0 + v1
        cnt = jnp.reshape(jnp.sum(cnts_ref[...], axis=0), (rb, 1))
        mean = ssum / jnp.maximum(cnt, 1.0)
        acc = lax.dot_general(mean, wl_ref[...], _DNUMS,
                              preferred_element_type=jnp.float32)
        out_ref[...] = acc + root_ref[...]

    grid = (-(-n // rb),)
    return pl.pallas_call(
        body,
        grid=grid,
        in_specs=[
            pl.BlockSpec((rb * d,), lambda i: (i,)),
            pl.BlockSpec((rb * d,), lambda i: (i,)),
            pl.BlockSpec((NW, rb), lambda i: (0, i)),
            pl.BlockSpec((rb, d), lambda i: (i, 0)),
            pl.BlockSpec((d, d), lambda i: (0, 0)),
        ],
        out_specs=pl.BlockSpec((rb, d), lambda i: (i, 0)),
        out_shape=jax.ShapeDtypeStruct((n, d), jnp.float32),
    )(s0, s1, cnts, root, wl)


def kernel(hidden_states, graph, aggl, W_l, b_l, W_r):
    x = hidden_states
    n, d = x.shape
    e = graph.shape[1]
    n_pad = -(-(n + 1) // 256) * 256

    if e % CH == 0:
        idx = graph
    else:
        pad = CH - e % CH
        idx = jnp.concatenate(
            [graph,
             jnp.stack([jnp.zeros((pad,), jnp.int32),
                        jnp.full((pad,), n, jnp.int32)])], axis=1)
    total_chunks = idx.shape[1] // CH

    root = _tc_root(x, W_r, b_l.reshape(1, d), n=n, d=d, rb=1024)
    sums0, sums1, cnts = _sc_aggregate(x.astype(jnp.bfloat16), idx,
                                       n_pad=n_pad, d=d,
                                       total_chunks=total_chunks)

    out = _tc_combine(jnp.reshape(sums0, (n_pad * d,)),
                      jnp.reshape(sums1, (n_pad * d,)),
                      cnts, root, W_l, n=n, d=d, rb=1024)
    return out
